# baseline (speedup 1.0000x reference)
import functools

import jax
import jax.numpy as jnp
from jax import lax
from jax.experimental import pallas as pl
from jax.experimental.pallas import tpu as pltpu
from jax.experimental.pallas import tpu_sc as plsc

B = 16384
L = 50
D = 32
NC = 2
NS = 16
LANES = 16
NW = NC * NS
ROWS_PER_W = B // NW
CHUNK = LANES
NCHUNK = ROWS_PER_W // CHUNK
GROUP = 10
NGROUP = L // GROUP


def _fm_body(uidx_hbm, iidx_hbm, uemb_hbm, iemb_hbm, out_hbm,
             iidx_v, uidx_v, uv, iv, out_v, isem, osem, gsem):
    wid = lax.axis_index("s") * NC + lax.axis_index("c")
    w_base = wid * ROWS_PER_W

    lane_iota = jax.lax.iota(jnp.int32, LANES)
    row_base = lane_iota * L

    def fire_idx(c, p):
        base = w_base + c * CHUNK
        pltpu.async_copy(iidx_hbm.at[pl.ds(base, CHUNK), :], iidx_v[p], isem[p])
        pltpu.async_copy(uidx_hbm.at[pl.ds(base, CHUNK)], uidx_v[p], isem[p])

    def wait_idx(p):
        pltpu.make_async_copy(iidx_hbm.at[pl.ds(0, CHUNK), :], iidx_v[p],
                              isem[p]).wait()
        pltpu.make_async_copy(uidx_hbm.at[pl.ds(0, CHUNK)], uidx_v[p],
                              isem[p]).wait()

    def fire_gathers(p):
        pltpu.async_copy(uemb_hbm.at[uidx_v[p]], uv[p], gsem)

    def drain_gathers(p):
        pltpu.make_async_copy(uemb_hbm.at[pl.ds(0, CHUNK), :], uv[p],
                              gsem).wait()

    def wait_out(p):
        pltpu.make_async_copy(out_v[p], out_hbm.at[pl.ds(0, CHUNK), :],
                              osem[p]).wait()

    def compute(c, p):
        for g in range(NGROUP):
            rows = [row_base + (g * GROUP + j) for j in range(GROUP)]

            def d_body(d, accs, rows=rows, p=p):
                dcol = jnp.full((LANES,), d, jnp.int32)
                u_d = plsc.load_gather(uv[p], [lane_iota, dcol])
                return tuple(
                    acc + u_d * plsc.load_gather(iv[p], [rows[j], dcol])
                    for j, acc in enumerate(accs))

            def d2_body(k, accs):
                return d_body(2 * k + 1, d_body(2 * k, accs))

            accs = lax.fori_loop(
                0, D // 2, d2_body,
                tuple(jnp.zeros((LANES,), jnp.float32) for _ in range(GROUP)))

            for j in range(GROUP):
                s = 1.0 / (1.0 + jnp.exp(-accs[j]))
                lcol = jnp.full((LANES,), g * GROUP + j, jnp.int32)
                plsc.store_scatter(out_v[p], [lane_iota, lcol], s)

        base = w_base + c * CHUNK
        pltpu.async_copy(out_v[p], out_hbm.at[pl.ds(base, CHUNK), :], osem[p])

    def half(c, p, fire_g_next, fire_idx2, do_out_wait):
        drain_gathers(p)
        if fire_g_next:
            wait_idx(1 - p)
            fire_gathers(1 - p)
        if fire_idx2:
            fire_idx(c + 2, p)
        if do_out_wait:
            wait_out(p)
        compute(c, p)

    fire_idx(0, 0)
    fire_idx(1, 1)
    wait_idx(0)
    fire_gathers(0)
    half(0, 0, True, True, False)
    half(1, 1, True, True, False)

    def pair_body(t, carry):
        half(2 * t, 0, True, True, True)
        half(2 * t + 1, 1, True, True, True)
        return carry

    lax.fori_loop(1, NCHUNK // 2 - 1, pair_body, 0)

    half(NCHUNK - 2, 0, True, False, True)
    half(NCHUNK - 1, 1, False, False, True)
    wait_out(0)
    wait_out(1)


_fm_kernel = functools.partial(
    pl.kernel,
    out_type=jax.ShapeDtypeStruct((B, L), jnp.float32),
    mesh=plsc.VectorSubcoreMesh(
        core_axis_name="c", subcore_axis_name="s",
        num_cores=NC, num_subcores=NS),
    compiler_params=pltpu.CompilerParams(
        needs_layout_passes=False, use_tc_tiling_on_sc=False),
    scratch_types=[
        [pltpu.VMEM((CHUNK, L), jnp.int32)] * 2,
        [pltpu.VMEM((CHUNK,), jnp.int32)] * 2,
        [pltpu.VMEM((CHUNK, D), jnp.float32)] * 2,
        [pltpu.VMEM((CHUNK * L, D), jnp.float32)] * 2,
        [pltpu.VMEM((CHUNK, L), jnp.float32)] * 2,
        [pltpu.SemaphoreType.DMA] * 2,
        [pltpu.SemaphoreType.DMA] * 2,
        pltpu.SemaphoreType.DMA,
    ],
)(_fm_body)


@jax.jit
def kernel(user_index, item_index, user_emb, item_emb):
    return _fm_kernel(user_index.astype(jnp.int32),
                      item_index.astype(jnp.int32),
                      user_emb, item_emb)


# speedup vs baseline: 1.2761x; 1.2761x over previous
import functools

import jax
import jax.numpy as jnp
from jax import lax
from jax.experimental import pallas as pl
from jax.experimental.pallas import tpu as pltpu
from jax.experimental.pallas import tpu_sc as plsc

B = 16384
L = 50
D = 32
NC = 2
NS = 16
LANES = 16
NW = NC * NS
ROWS_PER_W = B // NW
CHUNK = LANES
NCHUNK = ROWS_PER_W // CHUNK
TSLOT = 16 * 17


def _fm_body(uidx_hbm, iidx_hbm, uemb_hbm, iemb_hbm, out_hbm,
             iidx_v, uidx_v, uv, iv, tbuf, out_v, isem, osem, gsem):
    wid = lax.axis_index("s") * NC + lax.axis_index("c")
    w_base = wid * ROWS_PER_W

    lane_iota = jax.lax.iota(jnp.int32, LANES)

    def fire_idx(c, p):
        base = w_base + c * CHUNK
        pltpu.async_copy(iidx_hbm.at[pl.ds(base, CHUNK), :], iidx_v[p], isem[p])
        pltpu.async_copy(uidx_hbm.at[pl.ds(base, CHUNK)], uidx_v[p], isem[p])

    def wait_idx(p):
        pltpu.make_async_copy(iidx_hbm.at[pl.ds(0, CHUNK), :], iidx_v[p],
                              isem[p]).wait()
        pltpu.make_async_copy(uidx_hbm.at[pl.ds(0, CHUNK)], uidx_v[p],
                              isem[p]).wait()

    def fire_gathers(p):
        pltpu.async_copy(uemb_hbm.at[uidx_v[p]], uv[p], gsem)
        for j in range(CHUNK):
            pltpu.async_copy(iemb_hbm.at[iidx_v[p].at[j]],
                             iv[p].at[pl.ds(j * L, L), :], gsem)

    def drain_gathers(p):
        pltpu.make_async_copy(uemb_hbm.at[pl.ds(0, CHUNK), :], uv[p],
                              gsem).wait()
        pltpu.make_async_copy(iemb_hbm.at[pl.ds(0, CHUNK * L), :], iv[p],
                              gsem).wait()

    def wait_out(p):
        pltpu.make_async_copy(out_v[p], out_hbm.at[pl.ds(0, CHUNK), :],
                              osem[p]).wait()

    def compute(c, p):
        sc_base = lane_iota * (LANES + 1)

        def b_body(b, carry, p=p):
            u0 = uv[p][b, pl.ds(0, LANES)]
            u1 = uv[p][b, pl.ds(LANES, LANES)]
            rb = b * L
            for gi, l0 in enumerate((0, 16, 32, 34)):
                toff = (gi % 2) * TSLOT
                for i in range(LANES):
                    row = rb + l0 + i
                    prod = (u0 * iv[p][row, pl.ds(0, LANES)]
                            + u1 * iv[p][row, pl.ds(LANES, LANES)])
                    plsc.store_scatter(tbuf, [sc_base + (toff + i)], prod)
                parts = [tbuf[pl.ds(toff + d * (LANES + 1), LANES)]
                         for d in range(LANES)]
                while len(parts) > 1:
                    parts = [a + b2 for a, b2 in zip(parts[::2], parts[1::2])]
                s = 1.0 / (1.0 + jnp.exp(-parts[0]))
                out_v[p][b, pl.ds(l0, LANES)] = s
            return carry

        lax.fori_loop(0, CHUNK, b_body, 0)

        base = w_base + c * CHUNK
        pltpu.async_copy(out_v[p], out_hbm.at[pl.ds(base, CHUNK), :], osem[p])

    def half(c, p, fire_g_next, fire_idx2, do_out_wait):
        drain_gathers(p)
        if fire_g_next:
            wait_idx(1 - p)
            fire_gathers(1 - p)
        if fire_idx2:
            fire_idx(c + 2, p)
        if do_out_wait:
            wait_out(p)
        compute(c, p)

    fire_idx(0, 0)
    fire_idx(1, 1)
    wait_idx(0)
    fire_gathers(0)
    half(0, 0, True, True, False)
    half(1, 1, True, True, False)

    def pair_body(t, carry):
        half(2 * t, 0, True, True, True)
        half(2 * t + 1, 1, True, True, True)
        return carry

    lax.fori_loop(1, NCHUNK // 2 - 1, pair_body, 0)

    half(NCHUNK - 2, 0, True, False, True)
    half(NCHUNK - 1, 1, False, False, True)
    wait_out(0)
    wait_out(1)


_fm_kernel = functools.partial(
    pl.kernel,
    out_type=jax.ShapeDtypeStruct((B, L), jnp.float32),
    mesh=plsc.VectorSubcoreMesh(
        core_axis_name="c", subcore_axis_name="s",
        num_cores=NC, num_subcores=NS),
    compiler_params=pltpu.CompilerParams(
        needs_layout_passes=False, use_tc_tiling_on_sc=False),
    scratch_types=[
        [pltpu.VMEM((CHUNK, L), jnp.int32)] * 2,
        [pltpu.VMEM((CHUNK,), jnp.int32)] * 2,
        [pltpu.VMEM((CHUNK, D), jnp.float32)] * 2,
        [pltpu.VMEM((CHUNK * L, D), jnp.float32)] * 2,
        pltpu.VMEM((2 * 16 * 17,), jnp.float32),
        [pltpu.VMEM((CHUNK, L), jnp.float32)] * 2,
        [pltpu.SemaphoreType.DMA] * 2,
        [pltpu.SemaphoreType.DMA] * 2,
        pltpu.SemaphoreType.DMA,
    ],
)(_fm_body)


@jax.jit
def kernel(user_index, item_index, user_emb, item_emb):
    return _fm_kernel(user_index.astype(jnp.int32),
                      item_index.astype(jnp.int32),
                      user_emb, item_emb)


# speedup vs baseline: 1.2780x; 1.0015x over previous
import functools

import jax
import jax.numpy as jnp
from jax import lax
from jax.experimental import pallas as pl
from jax.experimental.pallas import tpu as pltpu
from jax.experimental.pallas import tpu_sc as plsc

B = 16384
L = 50
D = 32
NC = 2
NS = 16
LANES = 16
NW = NC * NS
ROWS_PER_W = B // NW
CHUNK = LANES
NCHUNK = ROWS_PER_W // CHUNK


def _fm_body(uidx_hbm, iidx_hbm, uemb_hbm, iemb_hbm, out_hbm,
             iidx_v, uidx_v, uv, iv, tbuf, out_v, isem, osem, gsem):
    tb = tbuf
    wid = lax.axis_index("s") * NC + lax.axis_index("c")
    w_base = wid * ROWS_PER_W

    lane_iota = jax.lax.iota(jnp.int32, LANES)

    def fire_idx(c, p):
        base = w_base + c * CHUNK
        pltpu.async_copy(iidx_hbm.at[pl.ds(base, CHUNK), :], iidx_v[p], isem[p])
        pltpu.async_copy(uidx_hbm.at[pl.ds(base, CHUNK)], uidx_v[p], isem[p])

    def wait_idx(p):
        pltpu.make_async_copy(iidx_hbm.at[pl.ds(0, CHUNK), :], iidx_v[p],
                              isem[p]).wait()
        pltpu.make_async_copy(uidx_hbm.at[pl.ds(0, CHUNK)], uidx_v[p],
                              isem[p]).wait()

    def fire_gathers(p):
        pltpu.async_copy(uemb_hbm.at[uidx_v[p]], uv[p], gsem)
        for j in range(CHUNK):
            pltpu.async_copy(iemb_hbm.at[iidx_v[p].at[j]],
                             iv[p].at[pl.ds(j * L, L), :], gsem)

    def drain_gathers(p):
        pltpu.make_async_copy(uemb_hbm.at[pl.ds(0, CHUNK), :], uv[p],
                              gsem).wait()
        pltpu.make_async_copy(iemb_hbm.at[pl.ds(0, CHUNK * L), :], iv[p],
                              gsem).wait()

    def wait_out(p):
        pltpu.make_async_copy(out_v[p], out_hbm.at[pl.ds(0, CHUNK), :],
                              osem[p]).wait()

    def compute(c, p):
        sc_base = lane_iota * (LANES + 1)

        def b_body(b, carry, p=p):
            u0 = uv[p][b, pl.ds(0, LANES)]
            u1 = uv[p][b, pl.ds(LANES, LANES)]
            rb = b * L
            for gi, l0 in enumerate((0, 16, 32, 34)):
                tg = tb[gi % 2]
                for i in range(LANES):
                    row = rb + l0 + i
                    prod = (u0 * iv[p][row, pl.ds(0, LANES)]
                            + u1 * iv[p][row, pl.ds(LANES, LANES)])
                    plsc.store_scatter(tg, [sc_base + i], prod)
                parts = [tg[pl.ds(d * (LANES + 1), LANES)]
                         for d in range(LANES)]
                while len(parts) > 1:
                    parts = [a + b2 for a, b2 in zip(parts[::2], parts[1::2])]
                s = 1.0 / (1.0 + jnp.exp(-parts[0]))
                out_v[p][b, pl.ds(l0, LANES)] = s
            return carry

        lax.fori_loop(0, CHUNK, b_body, 0)

        base = w_base + c * CHUNK
        pltpu.async_copy(out_v[p], out_hbm.at[pl.ds(base, CHUNK), :], osem[p])

    def half(c, p, fire_g_next, fire_idx2, do_out_wait):
        drain_gathers(p)
        if fire_g_next:
            wait_idx(1 - p)
            fire_gathers(1 - p)
        if fire_idx2:
            fire_idx(c + 2, p)
        if do_out_wait:
            wait_out(p)
        compute(c, p)

    fire_idx(0, 0)
    fire_idx(1, 1)
    wait_idx(0)
    fire_gathers(0)
    half(0, 0, True, True, False)
    half(1, 1, True, True, False)

    def pair_body(t, carry):
        half(2 * t, 0, True, True, True)
        half(2 * t + 1, 1, True, True, True)
        return carry

    lax.fori_loop(1, NCHUNK // 2 - 1, pair_body, 0)

    half(NCHUNK - 2, 0, True, False, True)
    half(NCHUNK - 1, 1, False, False, True)
    wait_out(0)
    wait_out(1)


_fm_kernel = functools.partial(
    pl.kernel,
    out_type=jax.ShapeDtypeStruct((B, L), jnp.float32),
    mesh=plsc.VectorSubcoreMesh(
        core_axis_name="c", subcore_axis_name="s",
        num_cores=NC, num_subcores=NS),
    compiler_params=pltpu.CompilerParams(
        needs_layout_passes=False, use_tc_tiling_on_sc=False),
    scratch_types=[
        [pltpu.VMEM((CHUNK, L), jnp.int32)] * 2,
        [pltpu.VMEM((CHUNK,), jnp.int32)] * 2,
        [pltpu.VMEM((CHUNK, D), jnp.float32)] * 2,
        [pltpu.VMEM((CHUNK * L, D), jnp.float32)] * 2,
        [pltpu.VMEM((16 * 17,), jnp.float32)] * 2,
        [pltpu.VMEM((CHUNK, L), jnp.float32)] * 2,
        [pltpu.SemaphoreType.DMA] * 2,
        [pltpu.SemaphoreType.DMA] * 2,
        pltpu.SemaphoreType.DMA,
    ],
)(_fm_body)


@jax.jit
def kernel(user_index, item_index, user_emb, item_emb):
    return _fm_kernel(user_index.astype(jnp.int32),
                      item_index.astype(jnp.int32),
                      user_emb, item_emb)


# speedup vs baseline: 1.5940x; 1.2473x over previous
import functools

import jax
import jax.numpy as jnp
from jax import lax
from jax.experimental import pallas as pl
from jax.experimental.pallas import tpu as pltpu
from jax.experimental.pallas import tpu_sc as plsc

B = 16384
L = 50
D = 32
NC = 2
NS = 16
LANES = 16
NW = NC * NS
ROWS_PER_W = B // NW
CHUNK = LANES
NCHUNK = ROWS_PER_W // CHUNK

_MESH = plsc.VectorSubcoreMesh(
    core_axis_name="c", subcore_axis_name="s",
    num_cores=NC, num_subcores=NS)


def _ugather_body(uidx_hbm, uemb_hbm, out_hbm,
                  uidx_v, blk, urow, isem, osem, bsem):
    wid = lax.axis_index("s") * NC + lax.axis_index("c")
    w_base = wid * ROWS_PER_W

    def fire_idx(c, p):
        base = w_base + c * CHUNK
        pltpu.async_copy(uidx_hbm.at[pl.ds(base, CHUNK)], uidx_v[p], isem[p])

    def wait_idx(p):
        pltpu.make_async_copy(uidx_hbm.at[pl.ds(0, CHUNK)], uidx_v[p],
                              isem[p]).wait()

    def fire_blocks(p):
        uvec = uidx_v[p][...]
        for i in range(CHUNK):
            u8 = pl.multiple_of((uvec[i] >> 3) << 3, 8)
            pltpu.async_copy(uemb_hbm.at[pl.ds(u8, 8), :], blk[p].at[i], bsem)

    def drain_blocks(p):
        for i in range(CHUNK):
            pltpu.make_async_copy(uemb_hbm.at[pl.ds(0, 8), :], blk[p].at[i],
                                  bsem).wait()

    def wait_out(p):
        pltpu.make_async_copy(urow[p], out_hbm.at[pl.ds(0, CHUNK * D)],
                              osem[p]).wait()

    def select_and_out(c, p):
        uvec = uidx_v[p][...]
        for i in range(CHUNK):
            m = uvec[i] & 7
            urow[p][pl.ds(i * D, LANES)] = blk[p][i, m, pl.ds(0, LANES)]
            urow[p][pl.ds(i * D + LANES, LANES)] = blk[p][i, m,
                                                          pl.ds(LANES, LANES)]
        base = (w_base + c * CHUNK) * D
        pltpu.async_copy(urow[p], out_hbm.at[pl.ds(base, CHUNK * D)], osem[p])

    def half(c, p, fire_next, fire_idx2, do_out_wait):
        drain_blocks(p)
        if fire_next:
            wait_idx(1 - p)
            fire_blocks(1 - p)
        if do_out_wait:
            wait_out(p)
        select_and_out(c, p)
        if fire_idx2:
            fire_idx(c + 2, p)

    fire_idx(0, 0)
    fire_idx(1, 1)
    wait_idx(0)
    fire_blocks(0)
    half(0, 0, True, True, False)
    half(1, 1, True, True, False)

    def pair_body(t, carry):
        half(2 * t, 0, True, True, True)
        half(2 * t + 1, 1, True, True, True)
        return carry

    lax.fori_loop(1, NCHUNK // 2 - 1, pair_body, 0)

    half(NCHUNK - 2, 0, True, False, True)
    half(NCHUNK - 1, 1, False, False, True)
    wait_out(0)
    wait_out(1)


_ugather_kernel = functools.partial(
    pl.kernel,
    out_type=jax.ShapeDtypeStruct((B * D,), jnp.float32),
    mesh=_MESH,
    compiler_params=pltpu.CompilerParams(needs_layout_passes=False),
    scratch_types=[
        [pltpu.VMEM((CHUNK,), jnp.int32)] * 2,
        [pltpu.VMEM((CHUNK, 8, D), jnp.float32)] * 2,
        [pltpu.VMEM((CHUNK * D,), jnp.float32)] * 2,
        [pltpu.SemaphoreType.DMA] * 2,
        [pltpu.SemaphoreType.DMA] * 2,
        pltpu.SemaphoreType.DMA,
    ],
)(_ugather_body)


def _fm_body(urows_hbm, iidx_hbm, iemb_hbm, out_hbm,
             iidx_v, uv, iv, tbuf, out_v, isem, osem, gsem):
    wid = lax.axis_index("s") * NC + lax.axis_index("c")
    w_base = wid * ROWS_PER_W

    lane_iota = jax.lax.iota(jnp.int32, LANES)

    def fire_idx(c, p):
        base = w_base + c * CHUNK
        pltpu.async_copy(iidx_hbm.at[pl.ds(base, CHUNK), :], iidx_v[p], isem[p])

    def wait_idx(p):
        pltpu.make_async_copy(iidx_hbm.at[pl.ds(0, CHUNK), :], iidx_v[p],
                              isem[p]).wait()

    def fire_gathers(c, p):
        base = (w_base + c * CHUNK) * D
        pltpu.async_copy(urows_hbm.at[pl.ds(base, CHUNK * D)], uv[p], gsem)
        for j in range(CHUNK):
            pltpu.async_copy(iemb_hbm.at[iidx_v[p].at[j]],
                             iv[p].at[pl.ds(j * L, L), :], gsem)

    def drain_gathers(p):
        pltpu.make_async_copy(urows_hbm.at[pl.ds(0, CHUNK * D)], uv[p],
                              gsem).wait()
        pltpu.make_async_copy(iemb_hbm.at[pl.ds(0, CHUNK * L), :], iv[p],
                              gsem).wait()

    def wait_out(p):
        pltpu.make_async_copy(out_v[p], out_hbm.at[pl.ds(0, CHUNK), :],
                              osem[p]).wait()

    def compute(c, p):
        sc_base = lane_iota * (LANES + 1)

        def b_body(b, carry, p=p):
            u0 = uv[p][pl.ds(b * D, LANES)]
            u1 = uv[p][pl.ds(b * D + LANES, LANES)]
            rb = b * L
            for gi, l0 in enumerate((0, 16, 32, 34)):
                tg = tbuf[gi % 2]
                for i in range(LANES):
                    row = rb + l0 + i
                    prod = (u0 * iv[p][row, pl.ds(0, LANES)]
                            + u1 * iv[p][row, pl.ds(LANES, LANES)])
                    plsc.store_scatter(tg, [sc_base + i], prod)
                parts = [tg[pl.ds(d * (LANES + 1), LANES)]
                         for d in range(LANES)]
                while len(parts) > 1:
                    parts = [a + b2 for a, b2 in zip(parts[::2], parts[1::2])]
                s = 1.0 / (1.0 + jnp.exp(-parts[0]))
                out_v[p][b, pl.ds(l0, LANES)] = s
            return carry

        lax.fori_loop(0, CHUNK, b_body, 0)

        base = w_base + c * CHUNK
        pltpu.async_copy(out_v[p], out_hbm.at[pl.ds(base, CHUNK), :], osem[p])

    def half(c, p, fire_g_next, fire_idx2, do_out_wait):
        drain_gathers(p)
        if fire_g_next:
            wait_idx(1 - p)
            fire_gathers(c + 1, 1 - p)
        if fire_idx2:
            fire_idx(c + 2, p)
        if do_out_wait:
            wait_out(p)
        compute(c, p)

    fire_idx(0, 0)
    fire_idx(1, 1)
    wait_idx(0)
    fire_gathers(0, 0)
    half(0, 0, True, True, False)
    half(1, 1, True, True, False)

    def pair_body(t, carry):
        half(2 * t, 0, True, True, True)
        half(2 * t + 1, 1, True, True, True)
        return carry

    lax.fori_loop(1, NCHUNK // 2 - 1, pair_body, 0)

    half(NCHUNK - 2, 0, True, False, True)
    half(NCHUNK - 1, 1, False, False, True)
    wait_out(0)
    wait_out(1)


_fm_kernel = functools.partial(
    pl.kernel,
    out_type=jax.ShapeDtypeStruct((B, L), jnp.float32),
    mesh=_MESH,
    compiler_params=pltpu.CompilerParams(
        needs_layout_passes=False, use_tc_tiling_on_sc=False),
    scratch_types=[
        [pltpu.VMEM((CHUNK, L), jnp.int32)] * 2,
        [pltpu.VMEM((CHUNK * D,), jnp.float32)] * 2,
        [pltpu.VMEM((CHUNK * L, D), jnp.float32)] * 2,
        [pltpu.VMEM((16 * 17,), jnp.float32)] * 2,
        [pltpu.VMEM((CHUNK, L), jnp.float32)] * 2,
        [pltpu.SemaphoreType.DMA] * 2,
        [pltpu.SemaphoreType.DMA] * 2,
        pltpu.SemaphoreType.DMA,
    ],
)(_fm_body)


@jax.jit
def kernel(user_index, item_index, user_emb, item_emb):
    urows = _ugather_kernel(user_index.astype(jnp.int32), user_emb)
    return _fm_kernel(urows, item_index.astype(jnp.int32), item_emb)


# speedup vs baseline: 1.5950x; 1.0006x over previous
import functools

import jax
import jax.numpy as jnp
from jax import lax
from jax.experimental import pallas as pl
from jax.experimental.pallas import tpu as pltpu
from jax.experimental.pallas import tpu_sc as plsc

B = 16384
L = 50
D = 32
NC = 2
NS = 16
LANES = 16
NW = NC * NS
ROWS_PER_W = B // NW
CHUNK = LANES
NCHUNK = ROWS_PER_W // CHUNK

_MESH = plsc.VectorSubcoreMesh(
    core_axis_name="c", subcore_axis_name="s",
    num_cores=NC, num_subcores=NS)


def _ugather_body(uidx_hbm, uemb_hbm, out_hbm,
                  uidx_v, blk, urow, isem, osem, bsem):
    wid = lax.axis_index("s") * NC + lax.axis_index("c")
    w_base = wid * ROWS_PER_W

    def fire_idx(c, p):
        base = w_base + c * CHUNK
        pltpu.async_copy(uidx_hbm.at[pl.ds(base, CHUNK)], uidx_v[p], isem[p])

    def wait_idx(p):
        pltpu.make_async_copy(uidx_hbm.at[pl.ds(0, CHUNK)], uidx_v[p],
                              isem[p]).wait()

    def fire_blocks(p):
        uvec = uidx_v[p][...]
        for i in range(CHUNK):
            u8 = pl.multiple_of((uvec[i] >> 3) << 3, 8)
            pltpu.async_copy(uemb_hbm.at[pl.ds(u8, 8), :], blk[p].at[i], bsem)

    def drain_blocks(p):
        for i in range(CHUNK):
            pltpu.make_async_copy(uemb_hbm.at[pl.ds(0, 8), :], blk[p].at[i],
                                  bsem).wait()

    def wait_out(p):
        pltpu.make_async_copy(urow[p], out_hbm.at[pl.ds(0, CHUNK * D)],
                              osem[p]).wait()

    def select_and_out(c, p):
        uvec = uidx_v[p][...]
        for i in range(CHUNK):
            m = uvec[i] & 7
            urow[p][pl.ds(i * D, LANES)] = blk[p][i, m, pl.ds(0, LANES)]
            urow[p][pl.ds(i * D + LANES, LANES)] = blk[p][i, m,
                                                          pl.ds(LANES, LANES)]
        base = (w_base + c * CHUNK) * D
        pltpu.async_copy(urow[p], out_hbm.at[pl.ds(base, CHUNK * D)], osem[p])

    def half(c, p, fire_next, fire_idx2, do_out_wait):
        drain_blocks(p)
        if fire_next:
            wait_idx(1 - p)
            fire_blocks(1 - p)
        if do_out_wait:
            wait_out(p)
        select_and_out(c, p)
        if fire_idx2:
            fire_idx(c + 2, p)

    fire_idx(0, 0)
    fire_idx(1, 1)
    wait_idx(0)
    fire_blocks(0)
    half(0, 0, True, True, False)
    half(1, 1, True, True, False)

    def pair_body(t, carry):
        half(2 * t, 0, True, True, True)
        half(2 * t + 1, 1, True, True, True)
        return carry

    lax.fori_loop(1, NCHUNK // 2 - 1, pair_body, 0)

    half(NCHUNK - 2, 0, True, False, True)
    half(NCHUNK - 1, 1, False, False, True)
    wait_out(0)
    wait_out(1)


_ugather_kernel = functools.partial(
    pl.kernel,
    out_type=jax.ShapeDtypeStruct((B * D,), jnp.float32),
    mesh=_MESH,
    scratch_types=[
        [pltpu.VMEM((CHUNK,), jnp.int32)] * 2,
        [pltpu.VMEM((CHUNK, 8, D), jnp.float32)] * 2,
        [pltpu.VMEM((CHUNK * D,), jnp.float32)] * 2,
        [pltpu.SemaphoreType.DMA] * 2,
        [pltpu.SemaphoreType.DMA] * 2,
        pltpu.SemaphoreType.DMA,
    ],
)(_ugather_body)


def _fm_body(urows_hbm, iidx_hbm, iemb_hbm, out_hbm,
             iidx_v, uv, iv, tbuf, out_v, isem, osem, gsem):
    wid = lax.axis_index("s") * NC + lax.axis_index("c")
    w_base = wid * ROWS_PER_W

    lane_iota = jax.lax.iota(jnp.int32, LANES)

    def fire_idx(c, p):
        base = w_base + c * CHUNK
        pltpu.async_copy(iidx_hbm.at[pl.ds(base, CHUNK), :], iidx_v[p], isem[p])

    def wait_idx(p):
        pltpu.make_async_copy(iidx_hbm.at[pl.ds(0, CHUNK), :], iidx_v[p],
                              isem[p]).wait()

    def fire_gathers(c, p):
        base = (w_base + c * CHUNK) * D
        pltpu.async_copy(urows_hbm.at[pl.ds(base, CHUNK * D)], uv[p], gsem)
        for j in range(CHUNK):
            pltpu.async_copy(iemb_hbm.at[iidx_v[p].at[j]],
                             iv[p].at[pl.ds(j * L, L), :], gsem)

    def drain_gathers(p):
        pltpu.make_async_copy(urows_hbm.at[pl.ds(0, CHUNK * D)], uv[p],
                              gsem).wait()
        pltpu.make_async_copy(iemb_hbm.at[pl.ds(0, CHUNK * L), :], iv[p],
                              gsem).wait()

    def wait_out(p):
        pltpu.make_async_copy(out_v[p], out_hbm.at[pl.ds(0, CHUNK), :],
                              osem[p]).wait()

    def compute(c, p):
        sc_base = lane_iota * (LANES + 1)

        def b_body(b, carry, p=p):
            u0 = uv[p][pl.ds(b * D, LANES)]
            u1 = uv[p][pl.ds(b * D + LANES, LANES)]
            rb = b * L
            for gi, l0 in enumerate((0, 16, 32, 34)):
                tg = tbuf[gi % 2]
                for i in range(LANES):
                    row = rb + l0 + i
                    prod = (u0 * iv[p][row, pl.ds(0, LANES)]
                            + u1 * iv[p][row, pl.ds(LANES, LANES)])
                    plsc.store_scatter(tg, [sc_base + i], prod)
                parts = [tg[pl.ds(d * (LANES + 1), LANES)]
                         for d in range(LANES)]
                while len(parts) > 1:
                    parts = [a + b2 for a, b2 in zip(parts[::2], parts[1::2])]
                s = 1.0 / (1.0 + jnp.exp(-parts[0]))
                out_v[p][b, pl.ds(l0, LANES)] = s
            return carry

        lax.fori_loop(0, CHUNK, b_body, 0)

        base = w_base + c * CHUNK
        pltpu.async_copy(out_v[p], out_hbm.at[pl.ds(base, CHUNK), :], osem[p])

    def half(c, p, fire_g_next, fire_idx2, do_out_wait):
        drain_gathers(p)
        if fire_g_next:
            wait_idx(1 - p)
            fire_gathers(c + 1, 1 - p)
        if fire_idx2:
            fire_idx(c + 2, p)
        if do_out_wait:
            wait_out(p)
        compute(c, p)

    fire_idx(0, 0)
    fire_idx(1, 1)
    wait_idx(0)
    fire_gathers(0, 0)
    half(0, 0, True, True, False)
    half(1, 1, True, True, False)

    def pair_body(t, carry):
        half(2 * t, 0, True, True, True)
        half(2 * t + 1, 1, True, True, True)
        return carry

    lax.fori_loop(1, NCHUNK // 2 - 1, pair_body, 0)

    half(NCHUNK - 2, 0, True, False, True)
    half(NCHUNK - 1, 1, False, False, True)
    wait_out(0)
    wait_out(1)


_fm_kernel = functools.partial(
    pl.kernel,
    out_type=jax.ShapeDtypeStruct((B, L), jnp.float32),
    mesh=_MESH,
    compiler_params=pltpu.CompilerParams(
        needs_layout_passes=False, use_tc_tiling_on_sc=False),
    scratch_types=[
        [pltpu.VMEM((CHUNK, L), jnp.int32)] * 2,
        [pltpu.VMEM((CHUNK * D,), jnp.float32)] * 2,
        [pltpu.VMEM((CHUNK * L, D), jnp.float32)] * 2,
        [pltpu.VMEM((16 * 17,), jnp.float32)] * 2,
        [pltpu.VMEM((CHUNK, L), jnp.float32)] * 2,
        [pltpu.SemaphoreType.DMA] * 2,
        [pltpu.SemaphoreType.DMA] * 2,
        pltpu.SemaphoreType.DMA,
    ],
)(_fm_body)


@jax.jit
def kernel(user_index, item_index, user_emb, item_emb):
    urows = _ugather_kernel(user_index.astype(jnp.int32), user_emb)
    return _fm_kernel(urows, item_index.astype(jnp.int32), item_emb)


# speedup vs baseline: 1.5976x; 1.0016x over previous
import functools

import jax
import jax.numpy as jnp
from jax import lax
from jax.experimental import pallas as pl
from jax.experimental.pallas import tpu as pltpu
from jax.experimental.pallas import tpu_sc as plsc

B = 16384
L = 50
D = 32
NC = 2
NS = 16
LANES = 16
NW = NC * NS
ROWS_PER_W = B // NW
CHUNK = LANES
NCHUNK = ROWS_PER_W // CHUNK

_MESH = plsc.VectorSubcoreMesh(
    core_axis_name="c", subcore_axis_name="s",
    num_cores=NC, num_subcores=NS)


def _ugather_body(uidx_hbm, uemb_hbm, out_hbm,
                  uidx_v, blk, urow, isem, osem, bsem):
    wid = lax.axis_index("s") * NC + lax.axis_index("c")
    w_base = wid * ROWS_PER_W

    def fire_idx(c, p):
        base = w_base + c * CHUNK
        pltpu.async_copy(uidx_hbm.at[pl.ds(base, CHUNK)], uidx_v[p], isem[p])

    def wait_idx(p):
        pltpu.make_async_copy(uidx_hbm.at[pl.ds(0, CHUNK)], uidx_v[p],
                              isem[p]).wait()

    def fire_blocks(p):
        uvec = uidx_v[p][...]
        for i in range(CHUNK):
            u8 = pl.multiple_of((uvec[i] >> 3) << 3, 8)
            pltpu.async_copy(uemb_hbm.at[pl.ds(u8, 8), :], blk[p].at[i], bsem)

    def drain_blocks(p):
        for i in range(CHUNK):
            pltpu.make_async_copy(uemb_hbm.at[pl.ds(0, 8), :], blk[p].at[i],
                                  bsem).wait()

    def wait_out(p):
        pltpu.make_async_copy(urow[p], out_hbm.at[pl.ds(0, CHUNK * D)],
                              osem[p]).wait()

    def select_and_out(c, p):
        uvec = uidx_v[p][...]
        for i in range(CHUNK):
            m = uvec[i] & 7
            urow[p][pl.ds(i * D, LANES)] = blk[p][i, m, pl.ds(0, LANES)]
            urow[p][pl.ds(i * D + LANES, LANES)] = blk[p][i, m,
                                                          pl.ds(LANES, LANES)]
        base = (w_base + c * CHUNK) * D
        pltpu.async_copy(urow[p], out_hbm.at[pl.ds(base, CHUNK * D)], osem[p])

    def half(c, p, fire_next, fire_idx2, do_out_wait):
        drain_blocks(p)
        if fire_next:
            wait_idx(1 - p)
            fire_blocks(1 - p)
        if do_out_wait:
            wait_out(p)
        select_and_out(c, p)
        if fire_idx2:
            fire_idx(c + 2, p)

    fire_idx(0, 0)
    fire_idx(1, 1)
    wait_idx(0)
    fire_blocks(0)
    half(0, 0, True, True, False)
    half(1, 1, True, True, False)

    def pair_body(t, carry):
        half(2 * t, 0, True, True, True)
        half(2 * t + 1, 1, True, True, True)
        return carry

    lax.fori_loop(1, NCHUNK // 2 - 1, pair_body, 0)

    half(NCHUNK - 2, 0, True, False, True)
    half(NCHUNK - 1, 1, False, False, True)
    wait_out(0)
    wait_out(1)


_ugather_kernel = functools.partial(
    pl.kernel,
    out_type=jax.ShapeDtypeStruct((B * D,), jnp.float32),
    mesh=_MESH,
    scratch_types=[
        [pltpu.VMEM((CHUNK,), jnp.int32)] * 2,
        [pltpu.VMEM((CHUNK, 8, D), jnp.float32)] * 2,
        [pltpu.VMEM((CHUNK * D,), jnp.float32)] * 2,
        [pltpu.SemaphoreType.DMA] * 2,
        [pltpu.SemaphoreType.DMA] * 2,
        pltpu.SemaphoreType.DMA,
    ],
)(_ugather_body)


def _fm_body(urows_hbm, iidx_hbm, iemb_hbm, out_hbm,
             iidx_v, uv, iv, tbuf, out_v, isem, osem, gsem):
    wid = lax.axis_index("s") * NC + lax.axis_index("c")
    w_base = wid * ROWS_PER_W

    lane_iota = jax.lax.iota(jnp.int32, LANES)

    def fire_idx(c, p):
        base = w_base + c * CHUNK
        pltpu.async_copy(iidx_hbm.at[pl.ds(base, CHUNK), :], iidx_v[p], isem[p])

    def wait_idx(p):
        pltpu.make_async_copy(iidx_hbm.at[pl.ds(0, CHUNK), :], iidx_v[p],
                              isem[p]).wait()

    def fire_gathers(c, p):
        base = (w_base + c * CHUNK) * D
        pltpu.async_copy(urows_hbm.at[pl.ds(base, CHUNK * D)], uv[p], gsem)
        for j in range(CHUNK):
            pltpu.async_copy(iemb_hbm.at[iidx_v[p].at[j]],
                             iv[p].at[pl.ds(j * L, L), :], gsem)

    def drain_gathers(p):
        pltpu.make_async_copy(urows_hbm.at[pl.ds(0, CHUNK * D)], uv[p],
                              gsem).wait()
        pltpu.make_async_copy(iemb_hbm.at[pl.ds(0, CHUNK * L), :], iv[p],
                              gsem).wait()

    def wait_out(p):
        pltpu.make_async_copy(out_v[p], out_hbm.at[pl.ds(0, CHUNK), :],
                              osem[p]).wait()

    def compute(c, p):
        sc_base = lane_iota * (LANES + 1)

        def b_body(b, carry, p=p):
            u0 = uv[p][pl.ds(b * D, LANES)]
            u1 = uv[p][pl.ds(b * D + LANES, LANES)]
            rb = b * L
            for gi, l0 in enumerate((0, 16, 32, 34)):
                tg = tbuf[gi]
                for i in range(LANES):
                    row = rb + l0 + i
                    prod = (u0 * iv[p][row, pl.ds(0, LANES)]
                            + u1 * iv[p][row, pl.ds(LANES, LANES)])
                    plsc.store_scatter(tg, [sc_base + i], prod)
            for gi, l0 in enumerate((0, 16, 32, 34)):
                tg = tbuf[gi]
                parts = [tg[pl.ds(d * (LANES + 1), LANES)]
                         for d in range(LANES)]
                while len(parts) > 1:
                    parts = [a + b2 for a, b2 in zip(parts[::2], parts[1::2])]
                s = 1.0 / (1.0 + jnp.exp(-parts[0]))
                out_v[p][b, pl.ds(l0, LANES)] = s
            return carry

        lax.fori_loop(0, CHUNK, b_body, 0)

        base = w_base + c * CHUNK
        pltpu.async_copy(out_v[p], out_hbm.at[pl.ds(base, CHUNK), :], osem[p])

    def half(c, p, fire_g_next, fire_idx2, do_out_wait):
        drain_gathers(p)
        if fire_g_next:
            wait_idx(1 - p)
            fire_gathers(c + 1, 1 - p)
        if fire_idx2:
            fire_idx(c + 2, p)
        if do_out_wait:
            wait_out(p)
        compute(c, p)

    fire_idx(0, 0)
    fire_idx(1, 1)
    wait_idx(0)
    fire_gathers(0, 0)
    half(0, 0, True, True, False)
    half(1, 1, True, True, False)

    def pair_body(t, carry):
        half(2 * t, 0, True, True, True)
        half(2 * t + 1, 1, True, True, True)
        return carry

    lax.fori_loop(1, NCHUNK // 2 - 1, pair_body, 0)

    half(NCHUNK - 2, 0, True, False, True)
    half(NCHUNK - 1, 1, False, False, True)
    wait_out(0)
    wait_out(1)


_fm_kernel = functools.partial(
    pl.kernel,
    out_type=jax.ShapeDtypeStruct((B, L), jnp.float32),
    mesh=_MESH,
    compiler_params=pltpu.CompilerParams(
        needs_layout_passes=False, use_tc_tiling_on_sc=False),
    scratch_types=[
        [pltpu.VMEM((CHUNK, L), jnp.int32)] * 2,
        [pltpu.VMEM((CHUNK * D,), jnp.float32)] * 2,
        [pltpu.VMEM((CHUNK * L, D), jnp.float32)] * 2,
        [pltpu.VMEM((16 * 17,), jnp.float32)] * 4,
        [pltpu.VMEM((CHUNK, L), jnp.float32)] * 2,
        [pltpu.SemaphoreType.DMA] * 2,
        [pltpu.SemaphoreType.DMA] * 2,
        pltpu.SemaphoreType.DMA,
    ],
)(_fm_body)


@jax.jit
def kernel(user_index, item_index, user_emb, item_emb):
    urows = _ugather_kernel(user_index.astype(jnp.int32), user_emb)
    return _fm_kernel(urows, item_index.astype(jnp.int32), item_emb)
